# Initial kernel scaffold; baseline (speedup 1.0000x reference)
import jax
import jax.numpy as jnp
from jax.experimental import pallas as pl


def kernel(mem, W1, W2, prob, arg_idx, target_idx):
    raise NotImplementedError("write your pallas kernel here")


# speedup vs baseline: 2.2522x; 2.2522x over previous
import functools

import jax
import jax.numpy as jnp
from jax import lax
from jax.experimental import pallas as pl
from jax.experimental.pallas import tpu as pltpu
from jax.experimental.pallas import tpu_sc as plsc

M, D, H, B = 100000, 512, 2048, 16384

_NC, _NS = 2, 16
_NW = _NC * _NS
_BPW = B // _NW
_GCH = 64
_SCH = 16

_mesh = plsc.VectorSubcoreMesh(core_axis_name="c", subcore_axis_name="s")


@functools.partial(
    pl.kernel,
    out_type=jax.ShapeDtypeStruct((B, D), jnp.float32),
    mesh=_mesh,
    scratch_types=[
        pltpu.VMEM((_BPW,), jnp.int32),
        pltpu.VMEM((_GCH, D), jnp.float32),
        pltpu.SemaphoreType.DMA,
    ],
)
def _sc_gather(mem_hbm, idx_hbm, out_hbm, idx_v, rows_v, sem):
    wid = lax.axis_index("s") * _NC + lax.axis_index("c")
    base = wid * _BPW
    pltpu.sync_copy(idx_hbm.at[pl.ds(base, _BPW)], idx_v)

    @pl.loop(0, _BPW // _GCH)
    def _chunk(c):
        off = c * _GCH
        pltpu.async_copy(
            mem_hbm.at[idx_v.at[pl.ds(off, _GCH)]], rows_v, sem
        ).wait()
        pltpu.sync_copy(rows_v, out_hbm.at[pl.ds(base + off, _GCH)])


_BM = 256
_GRID = B // _BM


def _mlp_body(x_ref, w1_ref, w2_ref, res_ref, p_ref, acc_ref):
    i = pl.program_id(0)
    h = jnp.maximum(
        jnp.dot(x_ref[...], w1_ref[...], preferred_element_type=jnp.float32),
        0.0,
    )
    r = jnp.dot(h, w2_ref[...], preferred_element_type=jnp.float32)
    res_ref[...] = r

    @pl.when(i == 0)
    def _init():
        acc_ref[0, 0] = 0.0

    acc_ref[0, 0] += jnp.sum(r)

    @pl.when(i == _GRID - 1)
    def _fin():
        p_ref[0, 0] = jax.nn.sigmoid(acc_ref[0, 0] / (B * D))


def _tc_mlp(gathered, W1, W2):
    return pl.pallas_call(
        _mlp_body,
        grid=(_GRID,),
        in_specs=[
            pl.BlockSpec((_BM, D), lambda i: (i, 0)),
            pl.BlockSpec((D, H), lambda i: (0, 0)),
            pl.BlockSpec((H, D), lambda i: (0, 0)),
        ],
        out_specs=[
            pl.BlockSpec((_BM, D), lambda i: (i, 0)),
            pl.BlockSpec(memory_space=pltpu.SMEM),
        ],
        out_shape=[
            jax.ShapeDtypeStruct((B, D), jnp.float32),
            jax.ShapeDtypeStruct((1, 1), jnp.float32),
        ],
        scratch_shapes=[pltpu.SMEM((1, 1), jnp.float32)],
    )(gathered, W1, W2)


_NCH = _BPW // _SCH


@functools.partial(
    pl.kernel,
    mesh=_mesh,
    scratch_types=[
        pltpu.VMEM((_NCH, _SCH), jnp.int32),
        pltpu.VMEM((_NCH, _SCH), jnp.int32),
        pltpu.VMEM((_SCH, D), jnp.float32),
        pltpu.SemaphoreType.DMA,
    ],
)
def _sc_scatter(out_ref, res_hbm, tgt2d_hbm, src2d_hbm, tgt_v, src_v, rows_v, sem):
    wid = lax.axis_index("s") * _NC + lax.axis_index("c")
    rbase = wid * _NCH
    pltpu.sync_copy(tgt2d_hbm.at[pl.ds(rbase, _NCH)], tgt_v)
    pltpu.sync_copy(src2d_hbm.at[pl.ds(rbase, _NCH)], src_v)

    @pl.loop(0, _NCH)
    def _chunk(c):
        pltpu.async_copy(res_hbm.at[src_v.at[c]], rows_v, sem).wait()
        pltpu.async_copy(rows_v, out_ref.at[tgt_v.at[c]], sem).wait()


def _winner_src(target_idx):
    order = jnp.argsort(target_idx, stable=True)
    ts = jnp.take(target_idx, order)
    pos = jnp.arange(B, dtype=jnp.int32)
    is_last = jnp.concatenate(
        [ts[:-1] != ts[1:], jnp.ones((1,), dtype=bool)]
    )
    marked = jnp.where(is_last, pos, B)
    last_pos = lax.cummin(marked[::-1])[::-1]
    winner_j = jnp.take(order, last_pos).astype(jnp.int32)
    return jnp.zeros((B,), jnp.int32).at[order].set(
        winner_j, unique_indices=True
    )


def kernel(mem, W1, W2, prob, arg_idx, target_idx):
    gathered = _sc_gather(mem, arg_idx)
    res, p = _tc_mlp(gathered, W1, W2)
    src = _winner_src(target_idx)
    mem_ref = jax.new_ref(mem)
    _sc_scatter(
        mem_ref,
        res,
        target_idx.reshape(_NW * _NCH, _SCH),
        src.reshape(_NW * _NCH, _SCH),
    )
    new_mem = mem_ref[...]
    new_prob = prob * p.reshape(1)
    return (new_mem, new_prob)
